# baseline (speedup 1.0000x reference)
import jax
import jax.numpy as jnp
from jax import lax
from jax.experimental import pallas as pl
from jax.experimental.pallas import tpu as pltpu
from jax.experimental.pallas import tpu_sc as plsc

N = 10000
K = 4
NITER = 10
E = 320000
NSUB = 16
EPS = 20000
CH = 2000
NCH = EPS // CH
GE = CH // 16
NPW = 640
NPB = 10240
NF = NPB * K
NFW = NPW * K
GB = NPW // 16

LN2 = 0.6931471805599453
F32 = jnp.float32
I32 = jnp.int32


_L1P = (5.7558723e-06, 0.99959326, -0.49503684, 0.30868933, -0.18376239,
        0.08709335, -0.028483821, 0.0055231405, -0.0004716229)


def _log1p(x):
    p = jnp.full((16,), _L1P[-1], F32)
    for c in _L1P[-2::-1]:
        p = p * x + c
    return p


def _iota16():
    return lax.iota(I32, 16)


def ZV():
    return jnp.zeros((16,), F32)


def _reduce_partials(aggh_r, f0, out_ref, tmps, sems):
    def _start(s, buf, sem):
        pltpu.make_async_copy(aggh_r.at[s, pl.ds(f0, NFW)], buf, sem).start()

    def _wait(buf, sem):
        pltpu.make_async_copy(aggh_r.at[0, pl.ds(f0, NFW)], buf, sem).wait()

    def _add(t):
        @plsc.parallel_loop(0, NFW // 64, step=1, unroll=2)
        def _acc(i):
            for u in range(4):
                o = i * 64 + u * 16
                out_ref[pl.ds(o, 16)] = out_ref[pl.ds(o, 16)] + t[pl.ds(o, 16)]

    pltpu.sync_copy(aggh_r.at[0, pl.ds(f0, NFW)], out_ref)
    for b in range(4):
        _start(1 + b, tmps[b], sems[b])

    def _quad(s4, c):
        s = 1 + 4 * s4
        for b in range(4):
            _wait(tmps[b], sems[b])
            _add(tmps[b])
            _start(jnp.minimum(s + 4 + b, NSUB - 1), tmps[b], sems[b])
        return c
    lax.fori_loop(0, 3, _quad, 0)
    for b in range(3):
        _wait(tmps[b], sems[b])
        _add(tmps[b])
    _wait(tmps[3], sems[3])


def _body(ec_r, b0_r, beta_r,
          s_r, q_r, aggh_r, part_r,
          bflat, aggf, eb0, eb1, tmp0, tmp1, tmp2, tmp3,
          agg_l, deg_l, newb, part_l, stage, beta_v,
          sem0, sem1, sem2, sem3):
    wid = lax.axis_index("s")
    f0 = wid * NFW
    cbase = wid * EPS
    sems = (sem0, sem1, sem2, sem3)
    tmps = (tmp0, tmp1, tmp2, tmp3)

    pltpu.sync_copy(b0_r, bflat)
    pltpu.sync_copy(beta_r, beta_v)
    beta_s = beta_v[...][0]

    def _edge_chunks(work, init):
        last = cbase + (NCH - 1) * CH

        def _start(off, buf, sem):
            pltpu.make_async_copy(ec_r.at[:, pl.ds(off, CH)], buf, sem).start()

        def _wait(buf, sem):
            pltpu.make_async_copy(ec_r.at[:, pl.ds(cbase, CH)], buf, sem).wait()

        _start(cbase, eb0, sem0)
        _start(cbase + CH, eb1, sem1)

        def _pair(c2, carry):
            c = cbase + 2 * c2 * CH
            _wait(eb0, sem0)
            carry = work(eb0, carry)
            _start(jnp.minimum(c + 2 * CH, last), eb0, sem0)
            _wait(eb1, sem1)
            carry = work(eb1, carry)
            _start(jnp.minimum(c + 3 * CH, last), eb1, sem1)
            return carry
        carry = lax.fori_loop(0, NCH // 2, _pair, init)
        _wait(eb0, sem0)
        _wait(eb1, sem1)
        return carry

    def _zero_aggf():
        @plsc.parallel_loop(0, NF // 64, step=1, unroll=2)
        def _z(i):
            for u in range(4):
                aggf[pl.ds(i * 64 + u * 16, 16)] = ZV()
    _zero_aggf()

    def _deg_work(ebuf, wacc):
        @plsc.parallel_loop(0, GE, step=1, unroll=5, carry=wacc)
        def _g(i, acc):
            wv = plsc.bitcast(ebuf[2, pl.ds(i * 16, 16)], F32)
            dv4 = ebuf[1, pl.ds(i * 16, 16)]
            for k in range(K):
                plsc.addupdate_scatter(aggf, [dv4 + k], wv)
            return acc + wv
        return _g
    wacc = _edge_chunks(_deg_work, ZV())
    stage[4, :] = wacc
    pltpu.sync_copy(stage, part_r.at[wid])
    pltpu.sync_copy(aggf, aggh_r.at[wid])
    _zero_aggf()
    plsc.subcore_barrier()

    _reduce_partials(aggh_r, f0, deg_l, tmps, sems)
    pltpu.sync_copy(part_r, part_l)
    wtot = ZV()
    for s in range(NSUB):
        wtot = wtot + part_l[s, 4, :]
    two_m = jnp.sum(wtot)
    inv2m = (1.0 / jnp.full((16,), two_m, F32))[0]

    def _f0(g, faccs):
        m4 = (g * 16 + _iota16()) * 4
        deg_v = plsc.load_gather(deg_l, [m4])
        out = []
        for k in range(K):
            bo = plsc.load_gather(bflat, [f0 + m4 + k])
            out.append(faccs[k] + deg_v * bo)
        return tuple(out)
    faccs = lax.fori_loop(0, GB, _f0, (ZV(), ZV(), ZV(), ZV()))
    for k in range(K):
        stage[k, :] = faccs[k]
    pltpu.sync_copy(stage, part_r.at[wid])
    plsc.subcore_barrier()
    pltpu.sync_copy(part_r, part_l)

    def _field():
        fs = []
        for k in range(K):
            acc = ZV()
            for s in range(NSUB):
                acc = acc + part_l[s, k, :]
            fs.append(jnp.sum(acc) * (-beta_s * inv2m))
        return tuple(fs)
    fld = _field()

    def _iter(t, fld):
        def _msg_work(ebuf, carry):
            @plsc.parallel_loop(0, GE, step=1, unroll=5)
            def _g(i):
                sv4 = ebuf[0, pl.ds(i * 16, 16)]
                dv4 = ebuf[1, pl.ds(i * 16, 16)]
                wv = plsc.bitcast(ebuf[2, pl.ds(i * 16, 16)], F32)
                ewv = jnp.exp(beta_s * wv) - 1.0
                for k in range(K):
                    gk = plsc.load_gather(bflat, [sv4 + k])
                    plsc.addupdate_scatter(aggf, [dv4 + k], _log1p(ewv * gk))
            return carry
        _edge_chunks(_msg_work, ZV())
        pltpu.sync_copy(aggf, aggh_r.at[wid])
        _zero_aggf()
        plsc.subcore_barrier()

        _reduce_partials(aggh_r, f0, agg_l, tmps, sems)

        @plsc.parallel_loop(0, GB, step=1, carry=(ZV(), ZV(), ZV(), ZV()))
        def _u(g, faccs):
            m4 = (g * 16 + _iota16()) * 4
            deg_v = plsc.load_gather(deg_l, [m4])
            logits = []
            for k in range(K):
                ak = plsc.load_gather(agg_l, [m4 + k])
                logits.append(ak + deg_v * fld[k])
            mx = jnp.maximum(jnp.maximum(logits[0], logits[1]),
                             jnp.maximum(logits[2], logits[3]))
            es = [jnp.exp(lk - mx) for lk in logits]
            inv = 1.0 / (es[0] + es[1] + es[2] + es[3])
            out = []
            for k in range(K):
                bo = plsc.load_gather(bflat, [f0 + m4 + k])
                bn = 0.5 * bo + 0.5 * es[k] * inv
                plsc.store_scatter(newb, [m4 + k], bn)
                out.append(faccs[k] + deg_v * bn)
            return tuple(out)
        faccs = _u
        pltpu.sync_copy(newb, s_r.at[pl.ds(f0, NFW)])
        for k in range(K):
            stage[k, :] = faccs[k]
        pltpu.sync_copy(stage, part_r.at[wid])
        plsc.subcore_barrier()

        pltpu.make_async_copy(s_r, bflat, sem0).start()
        pltpu.make_async_copy(part_r, part_l, sem1).start()
        pltpu.make_async_copy(part_r, part_l, sem1).wait()
        f_new = _field()
        pltpu.make_async_copy(s_r, bflat, sem0).wait()
        return f_new
    fld = lax.fori_loop(0, NITER, _iter, fld)

    def _q_work(ebuf, qacc):
        @plsc.parallel_loop(0, GE, step=1, unroll=4, carry=qacc)
        def _g(i, acc):
            sv4 = ebuf[0, pl.ds(i * 16, 16)]
            dv4 = ebuf[1, pl.ds(i * 16, 16)]
            wv = plsc.bitcast(ebuf[2, pl.ds(i * 16, 16)], F32)
            dot = ZV()
            for k in range(K):
                gs = plsc.load_gather(bflat, [sv4 + k])
                gd = plsc.load_gather(bflat, [dv4 + k])
                dot = dot + gs * gd
            return acc + wv * dot
        return _g
    qacc = _edge_chunks(_q_work, ZV())
    stage[5, :] = qacc
    pltpu.sync_copy(stage, part_r.at[wid])
    plsc.subcore_barrier()
    pltpu.sync_copy(part_r, part_l)

    qv = ZV()
    for s in range(NSUB):
        qv = qv + part_l[s, 5, :]
    qtot = jnp.sum(qv) * inv2m
    for k in range(K):
        acc = ZV()
        for s in range(NSUB):
            acc = acc + part_l[s, k, :]
        cd = jnp.sum(acc) * inv2m
        qtot = qtot - cd * cd

    @pl.when(wid == 0)
    def _():
        stage[6, :] = jnp.full((16,), qtot, F32)
        pltpu.sync_copy(stage.at[6], q_r)


@jax.jit
def _run(ec, b0, beta16):
    mesh = plsc.VectorSubcoreMesh(core_axis_name="c", subcore_axis_name="s",
                                  num_cores=1)
    f = pl.kernel(
        _body,
        out_type=[jax.ShapeDtypeStruct((NF,), F32),
                  jax.ShapeDtypeStruct((16,), F32),
                  jax.ShapeDtypeStruct((NSUB, NF), F32),
                  jax.ShapeDtypeStruct((NSUB, 8, 16), F32)],
        mesh=mesh,
        scratch_types=[
            pltpu.VMEM((NF,), F32),
            pltpu.VMEM((NF,), F32),
            pltpu.VMEM((3, CH), I32),
            pltpu.VMEM((3, CH), I32),
            pltpu.VMEM((NFW,), F32),
            pltpu.VMEM((NFW,), F32),
            pltpu.VMEM((NFW,), F32),
            pltpu.VMEM((NFW,), F32),
            pltpu.VMEM((NFW,), F32),
            pltpu.VMEM((NFW,), F32),
            pltpu.VMEM((NFW,), F32),
            pltpu.VMEM((NSUB, 8, 16), F32),
            pltpu.VMEM((8, 16), F32),
            pltpu.VMEM((16,), F32),
            pltpu.SemaphoreType.DMA,
            pltpu.SemaphoreType.DMA,
            pltpu.SemaphoreType.DMA,
            pltpu.SemaphoreType.DMA,
        ],
        compiler_params=pltpu.CompilerParams(needs_layout_passes=False,
                                             use_tc_tiling_on_sc=False),
    )
    return f(ec, b0, beta16)


def kernel(x, edge_index, edge_attr, beta):
    del x
    ec = jnp.stack([edge_index[0] * 4, edge_index[1] * 4,
                    lax.bitcast_convert_type(edge_attr[:, 0], I32)])

    init_logits = 0.1 * jnp.sin(
        jnp.arange(N * K, dtype=F32).reshape(N, K) * 0.37)
    b0 = jax.nn.softmax(init_logits, axis=-1)
    b0 = jnp.concatenate([b0.reshape(N * K), jnp.zeros((NF - N * K,), F32)])
    beta16 = jnp.full((16,), beta, F32)

    ec, b0, beta16 = lax.optimization_barrier((ec, b0, beta16))
    s_flat, q16, _, _ = _run(ec, b0, beta16)
    return (s_flat.reshape(NPB, K)[:N], q16[0])


# speedup vs baseline: 1.2658x; 1.2658x over previous
import jax
import jax.numpy as jnp
from jax import lax
from jax.experimental import pallas as pl
from jax.experimental.pallas import tpu as pltpu
from jax.experimental.pallas import tpu_sc as plsc

N = 10000
K = 4
NITER = 10
E = 320000
NSUB = 16
NW = 32
EPS = 10000
CH = 2000
NCH = EPS // CH
GE = CH // 16
NPW = 320
NPB = 10240
NF = NPB * K
NFW = NPW * K
GB = NPW // 16

LN2 = 0.6931471805599453
F32 = jnp.float32
I32 = jnp.int32


_L1P = (5.7558723e-06, 0.99959326, -0.49503684, 0.30868933, -0.18376239,
        0.08709335, -0.028483821, 0.0055231405, -0.0004716229)


def _log1p(x):
    p = jnp.full((16,), _L1P[-1], F32)
    for c in _L1P[-2::-1]:
        p = p * x + c
    return p


def _iota16():
    return lax.iota(I32, 16)


def ZV():
    return jnp.zeros((16,), F32)


def _reduce_partials(aggh_r, f0, out_ref, tmps, sems):
    def _start(s, buf, sem):
        pltpu.make_async_copy(aggh_r.at[s, pl.ds(f0, NFW)], buf, sem).start()

    def _wait(buf, sem):
        pltpu.make_async_copy(aggh_r.at[0, pl.ds(f0, NFW)], buf, sem).wait()

    def _add(t):
        @plsc.parallel_loop(0, NFW // 64, step=1, unroll=2)
        def _acc(i):
            for u in range(4):
                o = i * 64 + u * 16
                out_ref[pl.ds(o, 16)] = out_ref[pl.ds(o, 16)] + t[pl.ds(o, 16)]

    pltpu.sync_copy(aggh_r.at[0, pl.ds(f0, NFW)], out_ref)
    for b in range(4):
        _start(1 + b, tmps[b], sems[b])

    def _quad(s4, c):
        s = 1 + 4 * s4
        for b in range(4):
            _wait(tmps[b], sems[b])
            _add(tmps[b])
            _start(jnp.minimum(s + 4 + b, NW - 1), tmps[b], sems[b])
        return c
    lax.fori_loop(0, (NW - 4) // 4, _quad, 0)
    for b in range(3):
        _wait(tmps[b], sems[b])
        _add(tmps[b])
    _wait(tmps[3], sems[3])


def _body(ec_r, b0_r, beta_r,
          s_r, q_r, aggh_r, part_r, flag_r,
          bflat, aggf, eb0, eb1, tmp0, tmp1, tmp2, tmp3,
          agg_l, deg_l, newb, part_l, stage, beta_v, flg, fb,
          sem0, sem1, sem2, sem3):
    cid = lax.axis_index("c")
    sid = lax.axis_index("s")
    wid = cid * NSUB + sid
    f0 = wid * NFW
    cbase = wid * EPS
    sems = (sem0, sem1, sem2, sem3)
    tmps = (tmp0, tmp1, tmp2, tmp3)

    def _gb(ph):
        plsc.subcore_barrier()

        @pl.when(sid == 0)
        def _():
            phv = jnp.asarray(ph, I32)
            pat = (phv * jnp.int32(-1640531527) + _iota16() * 40503
                   + 747796405)
            slot = ph & 1
            flg[0, :] = jnp.zeros((16,), I32)
            flg[1, :] = pat
            pltpu.sync_copy(flg.at[0], flag_r.at[cid, 1 - slot])
            pltpu.sync_copy(flg.at[1], flag_r.at[cid, slot])

            def _poll(done):
                pltpu.sync_copy(flag_r.at[1 - cid, slot], fb)
                return jnp.all(fb[...] == pat)
            lax.while_loop(lambda d: jnp.logical_not(d), _poll,
                           jnp.bool_(False))
        plsc.subcore_barrier()

    pltpu.sync_copy(b0_r, bflat)
    pltpu.sync_copy(beta_r, beta_v)
    beta_s = beta_v[...][0]

    def _edge_chunks(work, init):
        last = cbase + (NCH - 1) * CH

        def _start(off, buf, sem):
            pltpu.make_async_copy(ec_r.at[:, pl.ds(off, CH)], buf, sem).start()

        def _wait(buf, sem):
            pltpu.make_async_copy(ec_r.at[:, pl.ds(cbase, CH)], buf, sem).wait()

        _start(cbase, eb0, sem0)
        _start(cbase + CH, eb1, sem1)

        def _pair(c2, carry):
            c = cbase + 2 * c2 * CH
            _wait(eb0, sem0)
            carry = work(eb0, carry)
            _start(jnp.minimum(c + 2 * CH, last), eb0, sem0)
            _wait(eb1, sem1)
            carry = work(eb1, carry)
            _start(jnp.minimum(c + 3 * CH, last), eb1, sem1)
            return carry
        carry = lax.fori_loop(0, NCH // 2, _pair, init)
        _wait(eb0, sem0)
        carry = work(eb0, carry)
        _wait(eb1, sem1)
        return carry

    def _zero_aggf():
        @plsc.parallel_loop(0, NF // 64, step=1, unroll=2)
        def _z(i):
            for u in range(4):
                aggf[pl.ds(i * 64 + u * 16, 16)] = ZV()
    _zero_aggf()

    def _deg_work(ebuf, wacc):
        @plsc.parallel_loop(0, GE, step=1, unroll=5, carry=wacc)
        def _g(i, acc):
            wv = plsc.bitcast(ebuf[2, pl.ds(i * 16, 16)], F32)
            dv4 = ebuf[1, pl.ds(i * 16, 16)]
            for k in range(K):
                plsc.addupdate_scatter(aggf, [dv4 + k], wv)
            return acc + wv
        return _g
    wacc = _edge_chunks(_deg_work, ZV())
    stage[4, :] = wacc
    pltpu.sync_copy(stage, part_r.at[wid])
    pltpu.sync_copy(aggf, aggh_r.at[wid])
    _zero_aggf()
    _gb(1)

    _reduce_partials(aggh_r, f0, deg_l, tmps, sems)
    pltpu.sync_copy(part_r, part_l)
    wtot = ZV()
    for s in range(NW):
        wtot = wtot + part_l[s, 4, :]
    two_m = jnp.sum(wtot)
    inv2m = (1.0 / jnp.full((16,), two_m, F32))[0]

    def _f0(g, faccs):
        m4 = (g * 16 + _iota16()) * 4
        deg_v = plsc.load_gather(deg_l, [m4])
        out = []
        for k in range(K):
            bo = plsc.load_gather(bflat, [f0 + m4 + k])
            out.append(faccs[k] + deg_v * bo)
        return tuple(out)
    faccs = lax.fori_loop(0, GB, _f0, (ZV(), ZV(), ZV(), ZV()))
    for k in range(K):
        stage[k, :] = faccs[k]
    pltpu.sync_copy(stage, part_r.at[wid])
    _gb(2)
    pltpu.sync_copy(part_r, part_l)

    def _field():
        fs = []
        for k in range(K):
            acc = ZV()
            for s in range(NW):
                acc = acc + part_l[s, k, :]
            fs.append(jnp.sum(acc) * (-beta_s * inv2m))
        return tuple(fs)
    fld = _field()

    def _iter(t, fld):
        def _msg_work(ebuf, carry):
            @plsc.parallel_loop(0, GE, step=1, unroll=5)
            def _g(i):
                sv4 = ebuf[0, pl.ds(i * 16, 16)]
                dv4 = ebuf[1, pl.ds(i * 16, 16)]
                wv = plsc.bitcast(ebuf[2, pl.ds(i * 16, 16)], F32)
                ewv = jnp.exp(beta_s * wv) - 1.0
                for k in range(K):
                    gk = plsc.load_gather(bflat, [sv4 + k])
                    plsc.addupdate_scatter(aggf, [dv4 + k], _log1p(ewv * gk))
            return carry
        _edge_chunks(_msg_work, ZV())
        pltpu.sync_copy(aggf, aggh_r.at[wid])
        _zero_aggf()
        _gb(3 + 2 * t)

        _reduce_partials(aggh_r, f0, agg_l, tmps, sems)

        @plsc.parallel_loop(0, GB, step=1, carry=(ZV(), ZV(), ZV(), ZV()))
        def _u(g, faccs):
            m4 = (g * 16 + _iota16()) * 4
            deg_v = plsc.load_gather(deg_l, [m4])
            logits = []
            for k in range(K):
                ak = plsc.load_gather(agg_l, [m4 + k])
                logits.append(ak + deg_v * fld[k])
            mx = jnp.maximum(jnp.maximum(logits[0], logits[1]),
                             jnp.maximum(logits[2], logits[3]))
            es = [jnp.exp(lk - mx) for lk in logits]
            inv = 1.0 / (es[0] + es[1] + es[2] + es[3])
            out = []
            for k in range(K):
                bo = plsc.load_gather(bflat, [f0 + m4 + k])
                bn = 0.5 * bo + 0.5 * es[k] * inv
                plsc.store_scatter(newb, [m4 + k], bn)
                out.append(faccs[k] + deg_v * bn)
            return tuple(out)
        faccs = _u
        pltpu.sync_copy(newb, s_r.at[pl.ds(f0, NFW)])
        for k in range(K):
            stage[k, :] = faccs[k]
        pltpu.sync_copy(stage, part_r.at[wid])
        _gb(4 + 2 * t)

        pltpu.make_async_copy(s_r, bflat, sem0).start()
        pltpu.make_async_copy(part_r, part_l, sem1).start()
        pltpu.make_async_copy(part_r, part_l, sem1).wait()
        f_new = _field()
        pltpu.make_async_copy(s_r, bflat, sem0).wait()
        return f_new
    fld = lax.fori_loop(0, NITER, _iter, fld)

    def _q_work(ebuf, qacc):
        @plsc.parallel_loop(0, GE, step=1, unroll=4, carry=qacc)
        def _g(i, acc):
            sv4 = ebuf[0, pl.ds(i * 16, 16)]
            dv4 = ebuf[1, pl.ds(i * 16, 16)]
            wv = plsc.bitcast(ebuf[2, pl.ds(i * 16, 16)], F32)
            dot = ZV()
            for k in range(K):
                gs = plsc.load_gather(bflat, [sv4 + k])
                gd = plsc.load_gather(bflat, [dv4 + k])
                dot = dot + gs * gd
            return acc + wv * dot
        return _g
    qacc = _edge_chunks(_q_work, ZV())
    stage[5, :] = qacc
    pltpu.sync_copy(stage, part_r.at[wid])
    _gb(25)
    pltpu.sync_copy(part_r, part_l)

    qv = ZV()
    for s in range(NW):
        qv = qv + part_l[s, 5, :]
    qtot = jnp.sum(qv) * inv2m
    for k in range(K):
        acc = ZV()
        for s in range(NW):
            acc = acc + part_l[s, k, :]
        cd = jnp.sum(acc) * inv2m
        qtot = qtot - cd * cd

    @pl.when(wid == 0)
    def _():
        stage[6, :] = jnp.full((16,), qtot, F32)
        pltpu.sync_copy(stage.at[6], q_r)
    _gb(26)


@jax.jit
def _run(ec, b0, beta16):
    mesh = plsc.VectorSubcoreMesh(core_axis_name="c", subcore_axis_name="s",
                                  num_cores=2)
    f = pl.kernel(
        _body,
        out_type=[jax.ShapeDtypeStruct((NF,), F32),
                  jax.ShapeDtypeStruct((16,), F32),
                  jax.ShapeDtypeStruct((NW, NF), F32),
                  jax.ShapeDtypeStruct((NW, 8, 16), F32),
                  jax.ShapeDtypeStruct((2, 2, 16), I32)],
        mesh=mesh,
        scratch_types=[
            pltpu.VMEM((NF,), F32),
            pltpu.VMEM((NF,), F32),
            pltpu.VMEM((3, CH), I32),
            pltpu.VMEM((3, CH), I32),
            pltpu.VMEM((NFW,), F32),
            pltpu.VMEM((NFW,), F32),
            pltpu.VMEM((NFW,), F32),
            pltpu.VMEM((NFW,), F32),
            pltpu.VMEM((NFW,), F32),
            pltpu.VMEM((NFW,), F32),
            pltpu.VMEM((NFW,), F32),
            pltpu.VMEM((NW, 8, 16), F32),
            pltpu.VMEM((8, 16), F32),
            pltpu.VMEM((16,), F32),
            pltpu.VMEM((2, 16), I32),
            pltpu.VMEM((16,), I32),
            pltpu.SemaphoreType.DMA,
            pltpu.SemaphoreType.DMA,
            pltpu.SemaphoreType.DMA,
            pltpu.SemaphoreType.DMA,
        ],
        compiler_params=pltpu.CompilerParams(needs_layout_passes=False,
                                             use_tc_tiling_on_sc=False),
    )
    return f(ec, b0, beta16)


def kernel(x, edge_index, edge_attr, beta):
    del x
    ec = jnp.stack([edge_index[0] * 4, edge_index[1] * 4,
                    lax.bitcast_convert_type(edge_attr[:, 0], I32)])

    init_logits = 0.1 * jnp.sin(
        jnp.arange(N * K, dtype=F32).reshape(N, K) * 0.37)
    b0 = jax.nn.softmax(init_logits, axis=-1)
    b0 = jnp.concatenate([b0.reshape(N * K), jnp.zeros((NF - N * K,), F32)])
    beta16 = jnp.full((16,), beta, F32)

    ec, b0, beta16 = lax.optimization_barrier((ec, b0, beta16))
    s_flat, q16, _, _, _ = _run(ec, b0, beta16)
    return (s_flat.reshape(NPB, K)[:N], q16[0])
